# baseline (speedup 1.0000x reference)
import functools

import jax
import jax.numpy as jnp
from jax import lax
from jax.experimental import pallas as pl
from jax.experimental.pallas import tpu as pltpu
from jax.experimental.pallas import tpu_sc as plsc

T, D, E, F = 2048, 2048, 8, 1408
TOPK = 2
TB = 256
B = 256
NT = 24
NR = NT * B
SLOTS = T * TOPK
NW = 32
SPW = SLOTS // NW
CH = 32
NCH = SPW // CH
TPW = T // NW
CHT = 8
NCHT = TPW // CHT


def _sc_mesh():
    return plsc.VectorSubcoreMesh(core_axis_name="c", subcore_axis_name="s")


def _router_body(x_ref, g_ref, idx_ref, w_ref, rank_ref, cnt_ref, acc_ref):
    i = pl.program_id(0)

    @pl.when(i == 0)
    def _():
        acc_ref[...] = jnp.zeros_like(acc_ref)

    logits = jnp.dot(x_ref[...], g_ref[...], preferred_element_type=jnp.float32)
    lane = lax.broadcasted_iota(jnp.int32, (TB, 128), 1)
    valid = lane < E
    lm = jnp.where(valid, logits, jnp.float32(-1e30))
    m = jnp.max(lm, axis=1, keepdims=True)
    ex = jnp.where(valid, jnp.exp(lm - m), 0.0)
    m1 = jnp.max(ex, axis=1, keepdims=True)
    i1 = jnp.min(jnp.where((ex == m1) & valid, lane, 128), axis=1, keepdims=True)
    ex2 = jnp.where(lane == i1, -1.0, ex)
    m2 = jnp.max(ex2, axis=1, keepdims=True)
    i2 = jnp.min(jnp.where((ex2 == m2) & valid, lane, 128), axis=1, keepdims=True)
    memb = ((lane == i1) | (lane == i2)).astype(jnp.float32)
    r = lax.broadcasted_iota(jnp.int32, (TB, TB), 0)
    c = lax.broadcasted_iota(jnp.int32, (TB, TB), 1)
    tri = (c < r).astype(jnp.float32)
    acc_row = acc_ref[0:1, :]
    ranks = jnp.dot(tri, memb, preferred_element_type=jnp.float32) + acc_row
    rank1 = jnp.sum(jnp.where(lane == i1, ranks, 0.0), axis=1, keepdims=True)
    rank2 = jnp.sum(jnp.where(lane == i2, ranks, 0.0), axis=1, keepdims=True)
    ws = m1 + m2
    idx_ref[...] = jnp.concatenate([i1, i2], axis=1)
    w_ref[...] = jnp.concatenate([m1 / ws, m2 / ws], axis=1)
    rank_ref[...] = jnp.concatenate([rank1, rank2], axis=1)
    newacc = acc_row + jnp.sum(memb, axis=0, keepdims=True)
    acc_ref[0:1, :] = newacc
    cnt_ref[...] = newacc


def _router(x, gpad):
    return pl.pallas_call(
        _router_body,
        grid=(T // TB,),
        in_specs=[
            pl.BlockSpec((TB, D), lambda i: (i, 0)),
            pl.BlockSpec((D, 128), lambda i: (0, 0)),
        ],
        out_specs=[
            pl.BlockSpec((TB, TOPK), lambda i: (i, 0)),
            pl.BlockSpec((TB, TOPK), lambda i: (i, 0)),
            pl.BlockSpec((TB, TOPK), lambda i: (i, 0)),
            pl.BlockSpec((1, 128), lambda i: (0, 0)),
        ],
        out_shape=[
            jax.ShapeDtypeStruct((T, TOPK), jnp.int32),
            jax.ShapeDtypeStruct((T, TOPK), jnp.float32),
            jax.ShapeDtypeStruct((T, TOPK), jnp.float32),
            jax.ShapeDtypeStruct((1, 128), jnp.float32),
        ],
        scratch_shapes=[pltpu.VMEM((8, 128), jnp.float32)],
    )(x, gpad)


def _plan_body(cnt_ref, idx_ref, rank_ref, pos_ref, eid_ref, act_ref):
    cnt = cnt_ref[...]
    padded = jnp.ceil(cnt / B) * B
    rr = lax.broadcasted_iota(jnp.int32, (128, 128), 0)
    cc = lax.broadcasted_iota(jnp.int32, (128, 128), 1)
    tri_u = (rr < cc).astype(jnp.float32)
    base = jnp.dot(padded, tri_u, preferred_element_type=jnp.float32)
    idx1 = idx_ref[:, 0:1]
    idx2 = idx_ref[:, 1:2]
    lane_t = lax.broadcasted_iota(jnp.int32, (T, 128), 1)
    b1 = jnp.sum(jnp.where(lane_t == idx1, base, 0.0), axis=1, keepdims=True)
    b2 = jnp.sum(jnp.where(lane_t == idx2, base, 0.0), axis=1, keepdims=True)
    pos0 = (b1 + rank_ref[:, 0:1]).astype(jnp.int32)
    pos1 = (b2 + rank_ref[:, 1:2]).astype(jnp.int32)
    pos_ref[...] = jnp.concatenate([pos0, pos1], axis=1)
    rows = lax.broadcasted_iota(jnp.int32, (32, 128), 0)
    lane32 = lax.broadcasted_iota(jnp.int32, (32, 128), 1)
    jb = (rows * B).astype(jnp.float32)
    ge = ((jb >= base) & (lane32 < E)).astype(jnp.int32)
    eid = jnp.maximum(jnp.sum(ge, axis=1, keepdims=True) - 1, 0)
    end_real = base + cnt
    sel_end = jnp.sum(jnp.where(lane32 == eid, end_real, 0.0), axis=1, keepdims=True)
    act = (jb[:, 0:1] < sel_end).astype(jnp.int32)
    eid_ref[...] = eid[0:NT, :]
    act_ref[...] = act[0:NT, :]


def _plan(cnt, idx_pair, rank_pair):
    return pl.pallas_call(
        _plan_body,
        out_shape=[
            jax.ShapeDtypeStruct((T, TOPK), jnp.int32),
            jax.ShapeDtypeStruct((NT, 1), jnp.int32),
            jax.ShapeDtypeStruct((NT, 1), jnp.int32),
        ],
    )(cnt, idx_pair, rank_pair)


def _dispatch_body(x_hbm, posq_hbm, wq_hbm, xs_hbm, ws_hbm,
                   idxp_v, idxt_v, buf_v, wbuf_v, sem):
    wid = lax.axis_index("s") * 2 + lax.axis_index("c")
    for cidx in range(NCH):
        qb = wid * SPW + cidx * CH
        pltpu.sync_copy(posq_hbm.at[pl.ds(qb, CH)], idxp_v)
        pltpu.sync_copy(wq_hbm.at[pl.ds(qb, CH)], wbuf_v)
        for h in range(CH // 16):
            tok = lax.shift_right_logical(qb + h * 16 + lax.iota(jnp.int32, 16), 1)
            idxt_v[pl.ds(h * 16, 16)] = tok
        pltpu.async_copy(x_hbm.at[idxt_v], buf_v, sem).wait()
        pltpu.async_copy(buf_v, xs_hbm.at[idxp_v], sem).wait()
        pltpu.async_copy(wbuf_v, ws_hbm.at[idxp_v], sem).wait()


def _dispatch(x, posq, wq):
    return pl.kernel(
        _dispatch_body,
        out_type=[
            jax.ShapeDtypeStruct((NR, D), jnp.float32),
            jax.ShapeDtypeStruct((NR, 128), jnp.float32),
        ],
        mesh=_sc_mesh(),
        scratch_types=[
            pltpu.VMEM((CH,), jnp.int32),
            pltpu.VMEM((CH,), jnp.int32),
            pltpu.VMEM((CH, D), jnp.float32),
            pltpu.VMEM((CH, 128), jnp.float32),
            pltpu.SemaphoreType.DMA,
        ],
    )(x, posq, wq)


DF = D * F
NCW = 4
CHF = DF // NCW


def _cast_body(src_ref, dst_ref):
    dst_ref[...] = src_ref[...].astype(jnp.bfloat16)


def _cast(w):
    flat = w.reshape(E, 1, DF)
    out = pl.pallas_call(
        _cast_body,
        grid=(E, NCW),
        in_specs=[pl.BlockSpec((1, 1, CHF), lambda e, c: (e, 0, c))],
        out_specs=pl.BlockSpec((1, 1, CHF), lambda e, c: (e, 0, c)),
        out_shape=jax.ShapeDtypeStruct((E, 1, DF), jnp.bfloat16),
    )(flat)
    return out.reshape(w.shape)


def _experts_body(eid_ref, act_ref, xs_ref, ws_ref, w1_ref, w3_ref, w2_ref,
                  ys_ref):
    j = pl.program_id(0)
    active = act_ref[j] > 0

    @pl.when(active)
    def _():
        xb = xs_ref[...].astype(jnp.bfloat16)
        h = jnp.dot(xb, w1_ref[0], preferred_element_type=jnp.float32)
        g = jnp.dot(xb, w3_ref[0], preferred_element_type=jnp.float32)
        a = (h * jax.nn.sigmoid(h) * g * ws_ref[:, 0:1]).astype(jnp.bfloat16)
        ys_ref[...] = jnp.dot(a, w2_ref[0], preferred_element_type=jnp.float32)

    @pl.when(jnp.logical_not(active))
    def _():
        ys_ref[...] = jnp.zeros_like(ys_ref)


def _experts(eid, act, xs, ws, w1, w3, w2):
    grid_spec = pltpu.PrefetchScalarGridSpec(
        num_scalar_prefetch=2,
        grid=(NT,),
        in_specs=[
            pl.BlockSpec((B, D), lambda j, eid, act: (j, 0)),
            pl.BlockSpec((B, 128), lambda j, eid, act: (j, 0)),
            pl.BlockSpec((1, D, F), lambda j, eid, act: (eid[j], 0, 0)),
            pl.BlockSpec((1, D, F), lambda j, eid, act: (eid[j], 0, 0)),
            pl.BlockSpec((1, F, D), lambda j, eid, act: (eid[j], 0, 0)),
        ],
        out_specs=pl.BlockSpec((B, D), lambda j, eid, act: (j, 0)),
    )
    return pl.pallas_call(
        _experts_body,
        grid_spec=grid_spec,
        out_shape=jax.ShapeDtypeStruct((NR, D), jnp.float32),
        compiler_params=pltpu.CompilerParams(
            dimension_semantics=("arbitrary",),
        ),
    )(eid, act, xs, ws, w1, w3, w2)


def _combine_body(ys_hbm, posq_hbm, out_hbm,
                  idx0_v, idx1_v, buf0_v, buf1_v, ob_v, sem0, sem1):
    wid = lax.axis_index("s") * 2 + lax.axis_index("c")
    idxs = (idx0_v, idx1_v)
    bufs = (buf0_v, buf1_v)
    sems = (sem0, sem1)

    def issue(c):
        tb = wid * TPW + c * CHT
        pltpu.sync_copy(posq_hbm.at[pl.ds(2 * tb, 2 * CHT)], idxs[c % 2])
        return pltpu.async_copy(ys_hbm.at[idxs[c % 2]], bufs[c % 2], sems[c % 2])

    cp = issue(0)
    for cidx in range(NCHT):
        cp.wait()
        if cidx + 1 < NCHT:
            cp = issue(cidx + 1)
        buf_v = bufs[cidx % 2]
        for rr in range(CHT):
            def body(i, carry):
                for u in range(8):
                    cs = (i * 8 + u) * 16
                    ob_v[rr, pl.ds(cs, 16)] = (
                        buf_v[2 * rr, pl.ds(cs, 16)]
                        + buf_v[2 * rr + 1, pl.ds(cs, 16)]
                    )
                return carry
            lax.fori_loop(0, D // 128, body, 0)
        pltpu.sync_copy(ob_v, out_hbm.at[pl.ds(wid * TPW + cidx * CHT, CHT)])


def _combine(ys, posq):
    return pl.kernel(
        _combine_body,
        out_type=jax.ShapeDtypeStruct((T, D), jnp.float32),
        mesh=_sc_mesh(),
        scratch_types=[
            pltpu.VMEM((2 * CHT,), jnp.int32),
            pltpu.VMEM((2 * CHT,), jnp.int32),
            pltpu.VMEM((2 * CHT, D), jnp.float32),
            pltpu.VMEM((2 * CHT, D), jnp.float32),
            pltpu.VMEM((CHT, D), jnp.float32),
            pltpu.SemaphoreType.DMA,
            pltpu.SemaphoreType.DMA,
        ],
    )(ys, posq)


def kernel(hidden_states, gate_w, w1, w3, w2):
    x = hidden_states
    gpad = jnp.pad(gate_w, ((0, 0), (0, 128 - E)))
    idx_pair, w_pair, rank_pair, cnt = _router(x, gpad)
    pos_pair, eid, act = _plan(cnt, idx_pair, rank_pair)
    posq = pos_pair.reshape(SLOTS)
    wq = jnp.broadcast_to(w_pair.reshape(SLOTS, 1), (SLOTS, 128))
    xs, ws = _dispatch(x, posq, wq)
    ys = _experts(eid.reshape(NT), act.reshape(NT), xs, ws,
                  _cast(w1), _cast(w3), _cast(w2))
    return _combine(ys, posq)


# speedup vs baseline: 5.0859x; 5.0859x over previous
import functools

import jax
import jax.numpy as jnp
from jax import lax
from jax.experimental import pallas as pl
from jax.experimental.pallas import tpu as pltpu
from jax.experimental.pallas import tpu_sc as plsc

T, D, E, F = 2048, 2048, 8, 1408
TOPK = 2
TB = 256
B = 256
NT = 24
NR = NT * B
SLOTS = T * TOPK
NW = 32
SPW = SLOTS // NW
CH = 32
NCH = SPW // CH
TPW = T // NW
CHT = 8
NCHT = TPW // CHT


def _sc_mesh():
    return plsc.VectorSubcoreMesh(core_axis_name="c", subcore_axis_name="s")


def _router_body(x_ref, g_ref, idx_ref, w_ref, rank_ref, cnt_ref, acc_ref):
    i = pl.program_id(0)

    @pl.when(i == 0)
    def _():
        acc_ref[...] = jnp.zeros_like(acc_ref)

    logits = jnp.dot(x_ref[...], g_ref[...], preferred_element_type=jnp.float32)
    lane = lax.broadcasted_iota(jnp.int32, (TB, 128), 1)
    valid = lane < E
    lm = jnp.where(valid, logits, jnp.float32(-1e30))
    m = jnp.max(lm, axis=1, keepdims=True)
    ex = jnp.where(valid, jnp.exp(lm - m), 0.0)
    m1 = jnp.max(ex, axis=1, keepdims=True)
    i1 = jnp.min(jnp.where((ex == m1) & valid, lane, 128), axis=1, keepdims=True)
    ex2 = jnp.where(lane == i1, -1.0, ex)
    m2 = jnp.max(ex2, axis=1, keepdims=True)
    i2 = jnp.min(jnp.where((ex2 == m2) & valid, lane, 128), axis=1, keepdims=True)
    memb = ((lane == i1) | (lane == i2)).astype(jnp.float32)
    r = lax.broadcasted_iota(jnp.int32, (TB, TB), 0)
    c = lax.broadcasted_iota(jnp.int32, (TB, TB), 1)
    tri = (c < r).astype(jnp.float32)
    acc_row = acc_ref[0:1, :]
    ranks = jnp.dot(tri, memb, preferred_element_type=jnp.float32) + acc_row
    rank1 = jnp.sum(jnp.where(lane == i1, ranks, 0.0), axis=1, keepdims=True)
    rank2 = jnp.sum(jnp.where(lane == i2, ranks, 0.0), axis=1, keepdims=True)
    ws = m1 + m2
    idx_ref[...] = jnp.concatenate([i1, i2], axis=1)
    w_ref[...] = jnp.concatenate([m1 / ws, m2 / ws], axis=1)
    rank_ref[...] = jnp.concatenate([rank1, rank2], axis=1)
    newacc = acc_row + jnp.sum(memb, axis=0, keepdims=True)
    acc_ref[0:1, :] = newacc
    cnt_ref[...] = newacc


def _router(x, gpad):
    return pl.pallas_call(
        _router_body,
        grid=(T // TB,),
        in_specs=[
            pl.BlockSpec((TB, D), lambda i: (i, 0)),
            pl.BlockSpec((D, 128), lambda i: (0, 0)),
        ],
        out_specs=[
            pl.BlockSpec((TB, TOPK), lambda i: (i, 0)),
            pl.BlockSpec((TB, TOPK), lambda i: (i, 0)),
            pl.BlockSpec((TB, TOPK), lambda i: (i, 0)),
            pl.BlockSpec((1, 128), lambda i: (0, 0)),
        ],
        out_shape=[
            jax.ShapeDtypeStruct((T, TOPK), jnp.int32),
            jax.ShapeDtypeStruct((T, TOPK), jnp.float32),
            jax.ShapeDtypeStruct((T, TOPK), jnp.float32),
            jax.ShapeDtypeStruct((1, 128), jnp.float32),
        ],
        scratch_shapes=[pltpu.VMEM((8, 128), jnp.float32)],
    )(x, gpad)


def _plan_body(cnt_ref, idx_ref, rank_ref, pos_ref, eid_ref, act_ref):
    cnt = cnt_ref[...]
    padded = jnp.ceil(cnt / B) * B
    rr = lax.broadcasted_iota(jnp.int32, (128, 128), 0)
    cc = lax.broadcasted_iota(jnp.int32, (128, 128), 1)
    tri_u = (rr < cc).astype(jnp.float32)
    base = jnp.dot(padded, tri_u, preferred_element_type=jnp.float32)
    idx1 = idx_ref[:, 0:1]
    idx2 = idx_ref[:, 1:2]
    lane_t = lax.broadcasted_iota(jnp.int32, (T, 128), 1)
    b1 = jnp.sum(jnp.where(lane_t == idx1, base, 0.0), axis=1, keepdims=True)
    b2 = jnp.sum(jnp.where(lane_t == idx2, base, 0.0), axis=1, keepdims=True)
    pos0 = (b1 + rank_ref[:, 0:1]).astype(jnp.int32)
    pos1 = (b2 + rank_ref[:, 1:2]).astype(jnp.int32)
    pos_ref[...] = jnp.concatenate([pos0, pos1], axis=1)
    rows = lax.broadcasted_iota(jnp.int32, (32, 128), 0)
    lane32 = lax.broadcasted_iota(jnp.int32, (32, 128), 1)
    jb = (rows * B).astype(jnp.float32)
    ge = ((jb >= base) & (lane32 < E)).astype(jnp.int32)
    eid = jnp.maximum(jnp.sum(ge, axis=1, keepdims=True) - 1, 0)
    end_real = base + cnt
    sel_end = jnp.sum(jnp.where(lane32 == eid, end_real, 0.0), axis=1, keepdims=True)
    act = (jb[:, 0:1] < sel_end).astype(jnp.int32)
    eid_ref[...] = eid[0:NT, :]
    act_ref[...] = act[0:NT, :]


def _plan(cnt, idx_pair, rank_pair):
    return pl.pallas_call(
        _plan_body,
        out_shape=[
            jax.ShapeDtypeStruct((T, TOPK), jnp.int32),
            jax.ShapeDtypeStruct((NT, 1), jnp.int32),
            jax.ShapeDtypeStruct((NT, 1), jnp.int32),
        ],
    )(cnt, idx_pair, rank_pair)


def _dispatch_body(x_hbm, posq_hbm, wq_hbm, xs_hbm, ws_hbm,
                   idxp_v, idxt_v, buf_v, wbuf_v, sem):
    wid = lax.axis_index("s") * 2 + lax.axis_index("c")
    for cidx in range(NCH):
        qb = wid * SPW + cidx * CH
        pltpu.sync_copy(posq_hbm.at[pl.ds(qb, CH)], idxp_v)
        pltpu.sync_copy(wq_hbm.at[pl.ds(qb, CH)], wbuf_v)
        for h in range(CH // 16):
            tok = lax.shift_right_logical(qb + h * 16 + lax.iota(jnp.int32, 16), 1)
            idxt_v[pl.ds(h * 16, 16)] = tok
        pltpu.async_copy(x_hbm.at[idxt_v], buf_v, sem).wait()
        pltpu.async_copy(buf_v, xs_hbm.at[idxp_v], sem).wait()
        pltpu.async_copy(wbuf_v, ws_hbm.at[idxp_v], sem).wait()


def _dispatch(x, posq, wq):
    return pl.kernel(
        _dispatch_body,
        out_type=[
            jax.ShapeDtypeStruct((NR, D), jnp.float32),
            jax.ShapeDtypeStruct((NR, 128), jnp.float32),
        ],
        mesh=_sc_mesh(),
        scratch_types=[
            pltpu.VMEM((CH,), jnp.int32),
            pltpu.VMEM((CH,), jnp.int32),
            pltpu.VMEM((CH, D), jnp.float32),
            pltpu.VMEM((CH, 128), jnp.float32),
            pltpu.SemaphoreType.DMA,
        ],
    )(x, posq, wq)


CAST_RB = 1024


def _cast_body(src_ref, dst_ref):
    dst_ref[...] = src_ref[...].astype(jnp.bfloat16)


def _cast(w):
    rows = w.shape[0] * w.shape[1]
    cols = w.shape[2]
    flat = w.reshape(rows, cols)
    out = pl.pallas_call(
        _cast_body,
        grid=(rows // CAST_RB,),
        in_specs=[pl.BlockSpec((CAST_RB, cols), lambda i: (i, 0))],
        out_specs=pl.BlockSpec((CAST_RB, cols), lambda i: (i, 0)),
        out_shape=jax.ShapeDtypeStruct((rows, cols), jnp.bfloat16),
    )(flat)
    return out.reshape(w.shape)


def _experts_body(eid_ref, act_ref, xs_ref, ws_ref, w1_ref, w3_ref, w2_ref,
                  ys_ref):
    j = pl.program_id(0)
    active = act_ref[j] > 0

    @pl.when(active)
    def _():
        xb = xs_ref[...].astype(jnp.bfloat16)
        h = jnp.dot(xb, w1_ref[0], preferred_element_type=jnp.float32)
        g = jnp.dot(xb, w3_ref[0], preferred_element_type=jnp.float32)
        a = (h * jax.nn.sigmoid(h) * g * ws_ref[:, 0:1]).astype(jnp.bfloat16)
        ys_ref[...] = jnp.dot(a, w2_ref[0], preferred_element_type=jnp.float32)

    @pl.when(jnp.logical_not(active))
    def _():
        ys_ref[...] = jnp.zeros_like(ys_ref)


def _experts(eid, act, xs, ws, w1, w3, w2):
    grid_spec = pltpu.PrefetchScalarGridSpec(
        num_scalar_prefetch=2,
        grid=(NT,),
        in_specs=[
            pl.BlockSpec((B, D), lambda j, eid, act: (j, 0)),
            pl.BlockSpec((B, 128), lambda j, eid, act: (j, 0)),
            pl.BlockSpec((1, D, F), lambda j, eid, act: (eid[j], 0, 0)),
            pl.BlockSpec((1, D, F), lambda j, eid, act: (eid[j], 0, 0)),
            pl.BlockSpec((1, F, D), lambda j, eid, act: (eid[j], 0, 0)),
        ],
        out_specs=pl.BlockSpec((B, D), lambda j, eid, act: (j, 0)),
    )
    return pl.pallas_call(
        _experts_body,
        grid_spec=grid_spec,
        out_shape=jax.ShapeDtypeStruct((NR, D), jnp.float32),
        compiler_params=pltpu.CompilerParams(
            dimension_semantics=("arbitrary",),
        ),
    )(eid, act, xs, ws, w1, w3, w2)


def _combine_body(ys_hbm, posq_hbm, out_hbm,
                  idx0_v, idx1_v, buf0_v, buf1_v, ob_v, sem0, sem1):
    wid = lax.axis_index("s") * 2 + lax.axis_index("c")
    idxs = (idx0_v, idx1_v)
    bufs = (buf0_v, buf1_v)
    sems = (sem0, sem1)

    def issue(c):
        tb = wid * TPW + c * CHT
        pltpu.sync_copy(posq_hbm.at[pl.ds(2 * tb, 2 * CHT)], idxs[c % 2])
        return pltpu.async_copy(ys_hbm.at[idxs[c % 2]], bufs[c % 2], sems[c % 2])

    cp = issue(0)
    for cidx in range(NCHT):
        cp.wait()
        if cidx + 1 < NCHT:
            cp = issue(cidx + 1)
        buf_v = bufs[cidx % 2]
        for rr in range(CHT):
            def body(i, carry):
                for u in range(8):
                    cs = (i * 8 + u) * 16
                    ob_v[rr, pl.ds(cs, 16)] = (
                        buf_v[2 * rr, pl.ds(cs, 16)]
                        + buf_v[2 * rr + 1, pl.ds(cs, 16)]
                    )
                return carry
            lax.fori_loop(0, D // 128, body, 0)
        pltpu.sync_copy(ob_v, out_hbm.at[pl.ds(wid * TPW + cidx * CHT, CHT)])


def _combine(ys, posq):
    return pl.kernel(
        _combine_body,
        out_type=jax.ShapeDtypeStruct((T, D), jnp.float32),
        mesh=_sc_mesh(),
        scratch_types=[
            pltpu.VMEM((2 * CHT,), jnp.int32),
            pltpu.VMEM((2 * CHT,), jnp.int32),
            pltpu.VMEM((2 * CHT, D), jnp.float32),
            pltpu.VMEM((2 * CHT, D), jnp.float32),
            pltpu.VMEM((CHT, D), jnp.float32),
            pltpu.SemaphoreType.DMA,
            pltpu.SemaphoreType.DMA,
        ],
    )(ys, posq)


def kernel(hidden_states, gate_w, w1, w3, w2):
    x = hidden_states
    gpad = jnp.pad(gate_w, ((0, 0), (0, 128 - E)))
    idx_pair, w_pair, rank_pair, cnt = _router(x, gpad)
    pos_pair, eid, act = _plan(cnt, idx_pair, rank_pair)
    posq = pos_pair.reshape(SLOTS)
    wq = jnp.broadcast_to(w_pair.reshape(SLOTS, 1), (SLOTS, 128))
    xs, ws = _dispatch(x, posq, wq)
    ys = _experts(eid.reshape(NT), act.reshape(NT), xs, ws,
                  _cast(w1), _cast(w3), _cast(w2))
    return _combine(ys, posq)


# speedup vs baseline: 5.5535x; 1.0920x over previous
import functools

import jax
import jax.numpy as jnp
from jax import lax
from jax.experimental import pallas as pl
from jax.experimental.pallas import tpu as pltpu
from jax.experimental.pallas import tpu_sc as plsc

T, D, E, F = 2048, 2048, 8, 1408
TOPK = 2
TB = 256
B = 256
NT = 24
NR = NT * B
SLOTS = T * TOPK
NW = 32
SPW = SLOTS // NW
CH = 32
NCH = SPW // CH
TPW = T // NW
CHT = 8
NCHT = TPW // CHT


def _sc_mesh():
    return plsc.VectorSubcoreMesh(core_axis_name="c", subcore_axis_name="s")


def _router_body(x_ref, g_ref, idx_ref, w_ref, rank_ref, cnt_ref, acc_ref):
    i = pl.program_id(0)

    @pl.when(i == 0)
    def _():
        acc_ref[...] = jnp.zeros_like(acc_ref)

    logits = jnp.dot(x_ref[...], g_ref[...], preferred_element_type=jnp.float32)
    lane = lax.broadcasted_iota(jnp.int32, (TB, 128), 1)
    valid = lane < E
    lm = jnp.where(valid, logits, jnp.float32(-1e30))
    m = jnp.max(lm, axis=1, keepdims=True)
    ex = jnp.where(valid, jnp.exp(lm - m), 0.0)
    m1 = jnp.max(ex, axis=1, keepdims=True)
    i1 = jnp.min(jnp.where((ex == m1) & valid, lane, 128), axis=1, keepdims=True)
    ex2 = jnp.where(lane == i1, -1.0, ex)
    m2 = jnp.max(ex2, axis=1, keepdims=True)
    i2 = jnp.min(jnp.where((ex2 == m2) & valid, lane, 128), axis=1, keepdims=True)
    memb = ((lane == i1) | (lane == i2)).astype(jnp.float32)
    r = lax.broadcasted_iota(jnp.int32, (TB, TB), 0)
    c = lax.broadcasted_iota(jnp.int32, (TB, TB), 1)
    tri = (c < r).astype(jnp.float32)
    acc_row = acc_ref[0:1, :]
    ranks = jnp.dot(tri, memb, preferred_element_type=jnp.float32) + acc_row
    rank1 = jnp.sum(jnp.where(lane == i1, ranks, 0.0), axis=1, keepdims=True)
    rank2 = jnp.sum(jnp.where(lane == i2, ranks, 0.0), axis=1, keepdims=True)
    ws = m1 + m2
    idx_ref[...] = jnp.concatenate([i1, i2], axis=1)
    w_ref[...] = jnp.concatenate([m1 / ws, m2 / ws], axis=1)
    rank_ref[...] = jnp.concatenate([rank1, rank2], axis=1)
    newacc = acc_row + jnp.sum(memb, axis=0, keepdims=True)
    acc_ref[0:1, :] = newacc
    cnt_ref[...] = newacc


def _router(x, gpad):
    return pl.pallas_call(
        _router_body,
        grid=(T // TB,),
        in_specs=[
            pl.BlockSpec((TB, D), lambda i: (i, 0)),
            pl.BlockSpec((D, 128), lambda i: (0, 0)),
        ],
        out_specs=[
            pl.BlockSpec((TB, TOPK), lambda i: (i, 0)),
            pl.BlockSpec((TB, TOPK), lambda i: (i, 0)),
            pl.BlockSpec((TB, TOPK), lambda i: (i, 0)),
            pl.BlockSpec((1, 128), lambda i: (0, 0)),
        ],
        out_shape=[
            jax.ShapeDtypeStruct((T, TOPK), jnp.int32),
            jax.ShapeDtypeStruct((T, TOPK), jnp.float32),
            jax.ShapeDtypeStruct((T, TOPK), jnp.float32),
            jax.ShapeDtypeStruct((1, 128), jnp.float32),
        ],
        scratch_shapes=[pltpu.VMEM((8, 128), jnp.float32)],
    )(x, gpad)


def _plan_body(cnt_ref, idx_ref, rank_ref, pos_ref, eid_ref, act_ref):
    cnt = cnt_ref[...]
    padded = jnp.ceil(cnt / B) * B
    rr = lax.broadcasted_iota(jnp.int32, (128, 128), 0)
    cc = lax.broadcasted_iota(jnp.int32, (128, 128), 1)
    tri_u = (rr < cc).astype(jnp.float32)
    base = jnp.dot(padded, tri_u, preferred_element_type=jnp.float32)
    idx1 = idx_ref[:, 0:1]
    idx2 = idx_ref[:, 1:2]
    lane_t = lax.broadcasted_iota(jnp.int32, (T, 128), 1)
    b1 = jnp.sum(jnp.where(lane_t == idx1, base, 0.0), axis=1, keepdims=True)
    b2 = jnp.sum(jnp.where(lane_t == idx2, base, 0.0), axis=1, keepdims=True)
    pos0 = (b1 + rank_ref[:, 0:1]).astype(jnp.int32)
    pos1 = (b2 + rank_ref[:, 1:2]).astype(jnp.int32)
    pos_ref[...] = jnp.concatenate([pos0, pos1], axis=1)
    rows = lax.broadcasted_iota(jnp.int32, (32, 128), 0)
    lane32 = lax.broadcasted_iota(jnp.int32, (32, 128), 1)
    jb = (rows * B).astype(jnp.float32)
    ge = ((jb >= base) & (lane32 < E)).astype(jnp.int32)
    eid = jnp.maximum(jnp.sum(ge, axis=1, keepdims=True) - 1, 0)
    end_real = base + cnt
    sel_end = jnp.sum(jnp.where(lane32 == eid, end_real, 0.0), axis=1, keepdims=True)
    act = (jb[:, 0:1] < sel_end).astype(jnp.int32)
    eid_ref[...] = eid[0:NT, :]
    act_ref[...] = act[0:NT, :]


def _plan(cnt, idx_pair, rank_pair):
    return pl.pallas_call(
        _plan_body,
        out_shape=[
            jax.ShapeDtypeStruct((T, TOPK), jnp.int32),
            jax.ShapeDtypeStruct((NT, 1), jnp.int32),
            jax.ShapeDtypeStruct((NT, 1), jnp.int32),
        ],
    )(cnt, idx_pair, rank_pair)


def _dispatch_body(x_hbm, posq_hbm, wq_hbm, xs_hbm, ws_hbm,
                   idxp_v, idxt_v, buf_v, wbuf_v, sem):
    wid = lax.axis_index("s") * 2 + lax.axis_index("c")
    for cidx in range(NCH):
        qb = wid * SPW + cidx * CH
        pltpu.sync_copy(posq_hbm.at[pl.ds(qb, CH)], idxp_v)
        pltpu.sync_copy(wq_hbm.at[pl.ds(qb, CH)], wbuf_v)
        for h in range(CH // 16):
            tok = lax.shift_right_logical(qb + h * 16 + lax.iota(jnp.int32, 16), 1)
            idxt_v[pl.ds(h * 16, 16)] = tok
        pltpu.async_copy(x_hbm.at[idxt_v], buf_v, sem).wait()
        pltpu.async_copy(buf_v, xs_hbm.at[idxp_v], sem).wait()
        pltpu.async_copy(wbuf_v, ws_hbm.at[idxp_v], sem).wait()


def _dispatch(x, posq, wq):
    return pl.kernel(
        _dispatch_body,
        out_type=[
            jax.ShapeDtypeStruct((NR, D), jnp.float32),
            jax.ShapeDtypeStruct((NR, 128), jnp.float32),
        ],
        mesh=_sc_mesh(),
        scratch_types=[
            pltpu.VMEM((CH,), jnp.int32),
            pltpu.VMEM((CH,), jnp.int32),
            pltpu.VMEM((CH, D), jnp.float32),
            pltpu.VMEM((CH, 128), jnp.float32),
            pltpu.SemaphoreType.DMA,
        ],
    )(x, posq, wq)


CAST_RB = 1024


def _cast_body(src_ref, dst_ref):
    dst_ref[...] = src_ref[...].astype(jnp.bfloat16)


def _cast(w):
    rows = w.shape[0] * w.shape[1]
    cols = w.shape[2]
    flat = w.reshape(rows, cols)
    out = pl.pallas_call(
        _cast_body,
        grid=(rows // CAST_RB,),
        in_specs=[pl.BlockSpec((CAST_RB, cols), lambda i: (i, 0))],
        out_specs=pl.BlockSpec((CAST_RB, cols), lambda i: (i, 0)),
        out_shape=jax.ShapeDtypeStruct((rows, cols), jnp.bfloat16),
    )(flat)
    return out.reshape(w.shape)


def _experts_body(eid_ref, act_ref, xs_ref, ws_ref, w1_ref, w3_ref, w2_ref,
                  ys_ref):
    j = pl.program_id(0)
    active = act_ref[j] > 0

    @pl.when(active)
    def _():
        xb = xs_ref[...].astype(jnp.bfloat16)
        h = jnp.dot(xb, w1_ref[0], preferred_element_type=jnp.float32)
        g = jnp.dot(xb, w3_ref[0], preferred_element_type=jnp.float32)
        a = h * jax.nn.sigmoid(h) * g * ws_ref[:, 0:1]
        ys_ref[...] = jnp.dot(a, w2_ref[0], preferred_element_type=jnp.float32,
                              precision=lax.Precision.DEFAULT)

    @pl.when(jnp.logical_not(active))
    def _():
        ys_ref[...] = jnp.zeros_like(ys_ref)


def _experts(eid, act, xs, ws, w1, w3, w2):
    grid_spec = pltpu.PrefetchScalarGridSpec(
        num_scalar_prefetch=2,
        grid=(NT,),
        in_specs=[
            pl.BlockSpec((B, D), lambda j, eid, act: (j, 0)),
            pl.BlockSpec((B, 128), lambda j, eid, act: (j, 0)),
            pl.BlockSpec((1, D, F), lambda j, eid, act: (eid[j], 0, 0)),
            pl.BlockSpec((1, D, F), lambda j, eid, act: (eid[j], 0, 0)),
            pl.BlockSpec((1, F, D), lambda j, eid, act: (eid[j], 0, 0)),
        ],
        out_specs=pl.BlockSpec((B, D), lambda j, eid, act: (j, 0)),
    )
    return pl.pallas_call(
        _experts_body,
        grid_spec=grid_spec,
        out_shape=jax.ShapeDtypeStruct((NR, D), jnp.float32),
        compiler_params=pltpu.CompilerParams(
            dimension_semantics=("arbitrary",),
        ),
    )(eid, act, xs, ws, w1, w3, w2)


def _combine_body(ys_hbm, posq_hbm, out_hbm,
                  idx0_v, idx1_v, buf0_v, buf1_v, ob_v, sem0, sem1):
    wid = lax.axis_index("s") * 2 + lax.axis_index("c")
    idxs = (idx0_v, idx1_v)
    bufs = (buf0_v, buf1_v)
    sems = (sem0, sem1)

    def issue(c):
        tb = wid * TPW + c * CHT
        pltpu.sync_copy(posq_hbm.at[pl.ds(2 * tb, 2 * CHT)], idxs[c % 2])
        return pltpu.async_copy(ys_hbm.at[idxs[c % 2]], bufs[c % 2], sems[c % 2])

    cp = issue(0)
    for cidx in range(NCHT):
        cp.wait()
        if cidx + 1 < NCHT:
            cp = issue(cidx + 1)
        buf_v = bufs[cidx % 2]
        for rr in range(CHT):
            def body(i, carry):
                for u in range(8):
                    cs = (i * 8 + u) * 16
                    ob_v[rr, pl.ds(cs, 16)] = (
                        buf_v[2 * rr, pl.ds(cs, 16)]
                        + buf_v[2 * rr + 1, pl.ds(cs, 16)]
                    )
                return carry
            lax.fori_loop(0, D // 128, body, 0)
        pltpu.sync_copy(ob_v, out_hbm.at[pl.ds(wid * TPW + cidx * CHT, CHT)])


def _combine(ys, posq):
    return pl.kernel(
        _combine_body,
        out_type=jax.ShapeDtypeStruct((T, D), jnp.float32),
        mesh=_sc_mesh(),
        scratch_types=[
            pltpu.VMEM((2 * CHT,), jnp.int32),
            pltpu.VMEM((2 * CHT,), jnp.int32),
            pltpu.VMEM((2 * CHT, D), jnp.float32),
            pltpu.VMEM((2 * CHT, D), jnp.float32),
            pltpu.VMEM((CHT, D), jnp.float32),
            pltpu.SemaphoreType.DMA,
            pltpu.SemaphoreType.DMA,
        ],
    )(ys, posq)


def kernel(hidden_states, gate_w, w1, w3, w2):
    x = hidden_states
    gpad = jnp.pad(gate_w, ((0, 0), (0, 128 - E)))
    idx_pair, w_pair, rank_pair, cnt = _router(x, gpad)
    pos_pair, eid, act = _plan(cnt, idx_pair, rank_pair)
    posq = pos_pair.reshape(SLOTS)
    wq = jnp.broadcast_to(w_pair.reshape(SLOTS, 1), (SLOTS, 128))
    xs, ws = _dispatch(x, posq, wq)
    ys = _experts(eid.reshape(NT), act.reshape(NT), xs, ws,
                  _cast(w1), _cast(w3), w2)
    return _combine(ys, posq)
